# baseline (speedup 1.0000x reference)
import functools

import jax
import jax.numpy as jnp
from jax import lax
from jax.experimental import pallas as pl
from jax.experimental.pallas import tpu as pltpu
from jax.experimental.pallas import tpu_sc as plsc

B, N, D, R, K, ORDER, H = 2, 96, 64, 32, 8, 3, 64
BI = 16
NW = 32
KP = 16
NEG = -1e30


def _silu(x):
    return x * jax.nn.sigmoid(x)


def _sc_topk(md, mk, rxf, ryf, rzf, rbf, node_s):
    apw = (B * N) // NW
    nc = N // 16
    mesh = plsc.VectorSubcoreMesh(core_axis_name="c", subcore_axis_name="s")
    f32 = jnp.float32
    i32 = jnp.int32

    @functools.partial(
        pl.kernel, mesh=mesh,
        out_type=[jax.ShapeDtypeStruct((B * N * KP,), i32)]
        + [jax.ShapeDtypeStruct((B * N * KP,), f32)] * 4
        + [jax.ShapeDtypeStruct((B * N * KP * R,), f32),
           jax.ShapeDtypeStruct((B * N * KP * D,), f32)],
        scratch_types=[pltpu.VMEM((N,), f32)] * 5
        + [pltpu.VMEM((KP,), i32)] + [pltpu.VMEM((KP,), f32)] * 4
        + [pltpu.VMEM((N * R,), f32), pltpu.VMEM((N * D,), f32),
           pltpu.VMEM((KP * R,), f32), pltpu.VMEM((KP * D,), f32)],
        compiler_params=pltpu.CompilerParams(needs_layout_passes=False),
    )
    def k(md_h, mk_h, rx_h, ry_h, rz_h, rbf_h, nd_h,
          oi_h, om_h, ox_h, oy_h, oz_h, orb_h, ond_h,
          md_v, mk_v, rx_v, ry_v, rz_v, oi_v, om_v, ox_v, oy_v, oz_v,
          rba_v, nda_v, rb_v, nd_v):
        wid = lax.axis_index("s") * 2 + lax.axis_index("c")
        iota = lax.iota(i32, 16)
        bw = (wid * apw) // N
        pltpu.sync_copy(nd_h.at[pl.ds(bw * N * D, N * D)], nda_v)

        def body(t, carry):
            a = wid * apw + t
            roff = a * N
            pltpu.sync_copy(md_h.at[pl.ds(roff, N)], md_v)
            pltpu.sync_copy(mk_h.at[pl.ds(roff, N)], mk_v)
            pltpu.sync_copy(rx_h.at[pl.ds(roff, N)], rx_v)
            pltpu.sync_copy(ry_h.at[pl.ds(roff, N)], ry_v)
            pltpu.sync_copy(rz_h.at[pl.ds(roff, N)], rz_v)
            v = [md_v[pl.ds(c * 16, 16)] for c in range(nc)]
            oidx = jnp.zeros((16,), i32)
            for r in range(K):
                m = v[0]
                for c in range(1, nc):
                    m = jnp.minimum(m, v[c])
                s = jnp.min(m)
                best = jnp.full((16,), 32767, i32)
                for c in range(nc):
                    eq = v[c] == s
                    pop = plsc.all_reduce_population_count(eq)
                    ffs = plsc.all_reduce_ffs(eq)
                    cand = jnp.where(pop > 0, c * 16 + ffs, 32767)
                    best = jnp.minimum(best, cand)
                oidx = jnp.where(iota == r, best, oidx)
                for c in range(nc):
                    v[c] = jnp.where(c * 16 + iota == best, 1e9, v[c])
            oi_v[...] = oidx
            om_v[...] = plsc.load_gather(mk_v, [oidx])
            ox_v[...] = plsc.load_gather(rx_v, [oidx])
            oy_v[...] = plsc.load_gather(ry_v, [oidx])
            oz_v[...] = plsc.load_gather(rz_v, [oidx])
            pltpu.sync_copy(rbf_h.at[pl.ds(a * N * R, N * R)], rba_v)
            o32 = oidx * R
            o64 = oidx * D
            for g in range(K * R // 16):
                base = o32.at[jnp.full((16,), g // (R // 16), i32)].get(
                    mode='promise_in_bounds')
                fi = base + (g % (R // 16)) * 16 + iota
                rb_v[pl.ds(g * 16, 16)] = plsc.load_gather(rba_v, [fi])
            for g in range(K * D // 16):
                base = o64.at[jnp.full((16,), g // (D // 16), i32)].get(
                    mode='promise_in_bounds')
                fi = base + (g % (D // 16)) * 16 + iota
                nd_v[pl.ds(g * 16, 16)] = plsc.load_gather(nda_v, [fi])
            ooff = a * KP
            pltpu.sync_copy(oi_v, oi_h.at[pl.ds(ooff, KP)])
            pltpu.sync_copy(om_v, om_h.at[pl.ds(ooff, KP)])
            pltpu.sync_copy(ox_v, ox_h.at[pl.ds(ooff, KP)])
            pltpu.sync_copy(oy_v, oy_h.at[pl.ds(ooff, KP)])
            pltpu.sync_copy(oz_v, oz_h.at[pl.ds(ooff, KP)])
            pltpu.sync_copy(rb_v, orb_h.at[pl.ds(a * KP * R, KP * R)])
            pltpu.sync_copy(nd_v, ond_h.at[pl.ds(a * KP * D, KP * D)])
            return carry

        lax.fori_loop(0, apw, body, 0)

    oi, om, ox, oy, oz, orb, ond = k(
        md.reshape(-1), mk.reshape(-1),
        rxf.reshape(-1), ryf.reshape(-1), rzf.reshape(-1),
        rbf.reshape(-1), node_s.reshape(-1))
    rs = (B, N, KP)
    return (oi.reshape(rs), om.reshape(rs), ox.reshape(rs),
            oy.reshape(rs), oz.reshape(rs),
            orb.reshape(B, N, KP * R), ond.reshape(B, N, KP * D))


def _fused_kernel(node_ref, mask_ref, rbf_ref, rx_ref, ry_ref, rz_ref,
                  oi_ref, om_ref, ox_ref, oy_ref, oz_ref, orb_ref, ond_ref,
                  ep_w1, ep_b1, ep_w2, ep_b2,
                  ktp, c6, cb0, tw2_2, tpb2t,
                  ts1_2, tsb1t, kts, w2p, ts_b2, bc,
                  tm_g, tm_b, tm_w1, tm_b1, tm_w2, tm_b2,
                  eg_w, eg_b, no_g, no_b, no_w, no_b2, en_g, en_b,
                  nd_out, es_out):
    f32 = jnp.float32
    i_blk = pl.program_id(1)
    node_b = node_ref[0]
    maskC = mask_ref[0]
    rbf = rbf_ref[0]
    rxC = rx_ref[0]
    ryC = ry_ref[0]
    rzC = rz_ref[0]

    idx = oi_ref[0][:, :K]
    tmask = om_ref[0][:, :K]
    crx = ox_ref[0][:, :K]
    cry = oy_ref[0][:, :K]
    crz = oz_ref[0][:, :K]
    crbf = orb_ref[0]
    cnd = ond_ref[0]

    logit_parts = []
    twcf_parts = []
    for p in range(K // 2):
        s0 = 2 * p
        cosp = (rxC * crx[:, None, s0:s0 + 2] +
                ryC * cry[:, None, s0:s0 + 2] +
                rzC * crz[:, None, s0:s0 + 2])
        cosp = jnp.clip(cosp, -1.0 + 1e-6, 1.0 - 1e-6)
        c2p = cosp * cosp
        c3p = c2p * cosp
        xp = jnp.concatenate([cosp, c2p, c3p], axis=2)
        radp = crbf[:, 2 * R * p:2 * R * (p + 1)] @ ktp[...]
        bslice = radp + cb0[:, 128 * p:128 * (p + 1)]
        thp = (xp.reshape(BI * N, 6) @ c6[...]).reshape(BI, N, 2 * H) \
            + bslice[:, None, :]
        thp = _silu(thp)
        twp = thp.reshape(BI * N, 2 * H) @ tw2_2[...] + tpb2t[...]
        cfp = cnd[:, 128 * p:128 * (p + 1)]
        cfs = cfp @ kts[...]
        shp = _silu((twp @ ts1_2[...] + tsb1t[...]).reshape(BI, N, 2 * H)
                    + cfs[:, None, :])
        logit_parts.append(shp.reshape(BI * N, 2 * H) @ w2p[...])
        twcf_parts.append(twp.reshape(BI, N, 2 * H) * cfp[:, None, :])

    logits = jnp.concatenate(logit_parts, axis=1).reshape(BI, N, K) \
        + ts_b2[...].reshape(1, 1, 1)

    iota_jn = jax.lax.broadcasted_iota(jnp.int32, (BI, N, K), 1)
    pm = maskC * tmask[:, None, :]
    pm = jnp.where(idx[:, None, :] == iota_jn, 0.0, pm)

    logits = jnp.where(pm <= 0.0, NEG, logits)
    lmax = jnp.max(logits, axis=2, keepdims=True)
    e = jnp.exp(logits - lmax)
    attn = e / jnp.sum(e, axis=2, keepdims=True)
    attn = jnp.where(pm > 0.0, attn, 0.0)

    attnb = attn.reshape(BI * N, K) @ bc[...]
    pmb = pm.reshape(BI * N, K) @ bc[...]
    attnb = attnb.reshape(BI, N, K * D)
    pmb = pmb.reshape(BI, N, K * D)

    t_attn = jnp.zeros((BI, N, D), f32)
    t_max = jnp.full((BI, N, D), NEG, f32)
    for p in range(K // 2):
        twcf = twcf_parts[p]
        wp = twcf * attnb[:, :, 128 * p:128 * (p + 1)]
        t_attn = t_attn + wp[:, :, :D] + wp[:, :, D:]
        mp = jnp.where(pmb[:, :, 128 * p:128 * (p + 1)] <= 0.0, NEG, twcf)
        t_max = jnp.maximum(t_max, jnp.maximum(mp[:, :, :D], mp[:, :, D:]))
    t_max = jnp.where(t_max <= NEG * 0.5, 0.0, t_max)

    mi = jnp.concatenate([t_attn, t_max], axis=2)
    mu = jnp.mean(mi, axis=2, keepdims=True)
    mv = jnp.mean((mi - mu) ** 2, axis=2, keepdims=True)
    mi = (mi - mu) * (1.0 / jnp.sqrt(mv + 1e-5)) * tm_g[...].reshape(1, 1, 2 * D) \
        + tm_b[...].reshape(1, 1, 2 * D)
    mh = _silu(mi.reshape(BI * N, 2 * D) @ tm_w1[...] + tm_b1[...])
    ctx = mh @ tm_w2[...] + tm_b2[...]

    node_i = node_ref[0, pl.ds(i_blk * BI, BI), :]
    hi = node_i @ ep_w1[:D, :]
    dstW = node_b @ ep_w1[D:2 * D, :]
    rbfW1 = (rbf.reshape(BI * N, R) @ ep_w1[2 * D:, :]).reshape(BI, N, H)
    eh = _silu(hi[:, None, :] + dstW[None, :, :] + rbfW1
               + ep_b1[...].reshape(1, 1, H))
    eb = (eh.reshape(BI * N, H) @ ep_w2[...] + ep_b2[...]).reshape(BI, N, D)
    mask3 = maskC
    eb = eb * mask3

    ef = eb + ctx.reshape(BI, N, D)
    emu = jnp.mean(ef, axis=2, keepdims=True)
    ev = jnp.mean((ef - emu) ** 2, axis=2, keepdims=True)
    ef = (ef - emu) * (1.0 / jnp.sqrt(ev + 1e-5)) * en_g[...].reshape(1, 1, D) \
        + en_b[...].reshape(1, 1, D)
    gate = jax.nn.sigmoid(ef.reshape(BI * N, D) @ eg_w[...]
                          + eg_b[...]).reshape(BI, N, D)
    ef = gate * ef

    ns = jnp.sum(ef * mask3, axis=1)
    es = jnp.sum(ef, axis=1)

    nmu = jnp.mean(ns, axis=1, keepdims=True)
    nv = jnp.mean((ns - nmu) ** 2, axis=1, keepdims=True)
    nd = (ns - nmu) * (1.0 / jnp.sqrt(nv + 1e-5)) * no_g[...] + no_b[...]
    nd = nd @ no_w[...] + no_b2[...]

    nd_out[0] = nd
    es_out[0] = es


def kernel(node_s, dist, rbf, r_hat, access_mask, params):
    p = params
    f32 = jnp.float32
    maskf = access_mask.astype(f32)
    maxd = jnp.maximum(dist.max(axis=(1, 2), keepdims=True), 1.0) + 1.0
    md = jnp.where(access_mask, dist, maxd)
    rx = r_hat[..., 0]
    ry = r_hat[..., 1]
    rz = r_hat[..., 2]

    oi, om, ox, oy, oz, orb, ond = _sc_topk(md, maskf, rx, ry, rz, rbf, node_s)

    def row2(v):
        return v.reshape(1, -1)

    w = p['tp_w1']
    tp_c = jnp.stack([w[0] - 0.5 * w[2], w[1] - 1.5 * w[3],
                      1.5 * w[2], 2.5 * w[3]], axis=0)
    eye2 = jnp.eye(2, dtype=f32)
    eyeK = jnp.eye(K, dtype=f32)
    c6 = jnp.einsum('qr,th->tqrh', eye2, tp_c[1:]).reshape(6, 2 * H)
    cb0 = jnp.tile(tp_c[0] + p['tp_b1'], 2 * K // 2).reshape(1, K * H)
    tw2_2 = jnp.einsum('qr,hd->qhrd', eye2, p['tp_w2']).reshape(2 * H, 2 * H)
    tpb2t = jnp.tile(p['tp_b2'], 2).reshape(1, 2 * H)
    ts1_2 = jnp.einsum('qr,dh->qdrh', eye2, p['ts_w1'][:D]).reshape(2 * D, 2 * H)
    tsb1t = jnp.tile(p['ts_b1'], 2).reshape(1, 2 * H)
    w2p = jnp.einsum('qr,ho->qhro', eye2, p['ts_w2']).reshape(2 * H, 2)
    bc = jnp.einsum('kl,d->kld', eyeK, jnp.ones((D,), f32)).reshape(K, K * D)
    ktp = jnp.einsum('qr,ah->qarh', eye2,
                     p['tp_w1'][ORDER + 1:]).reshape(2 * R, 2 * H)
    kts = jnp.einsum('qr,dh->qdrh', eye2, p['ts_w1'][D:]).reshape(2 * D, 2 * H)

    maskC = maskf.reshape(B, N, N, 1)
    rxC = rx.reshape(B, N, N, 1)
    ryC = ry.reshape(B, N, N, 1)
    rzC = rz.reshape(B, N, N, 1)

    args = (node_s, maskC, rbf, rxC, ryC, rzC, oi, om, ox, oy, oz, orb, ond,
            p['ep_w1'], row2(p['ep_b1']), p['ep_w2'], row2(p['ep_b2']),
            ktp, c6, cb0, tw2_2, tpb2t,
            ts1_2, tsb1t, kts, w2p, row2(p['ts_b2']), bc,
            row2(p['tm_g']), row2(p['tm_b']),
            p['tm_w1'], row2(p['tm_b1']), p['tm_w2'], row2(p['tm_b2']),
            p['eg_w'], row2(p['eg_b']), row2(p['no_g']), row2(p['no_b']),
            p['no_w'], row2(p['no_b2']), row2(p['en_g']), row2(p['en_b']))

    def full(a):
        return pl.BlockSpec(a.shape, lambda b, i: (0,) * a.ndim)

    row_specs = [
        pl.BlockSpec((1, N, D), lambda b, i: (b, 0, 0)),
        pl.BlockSpec((1, BI, N, 1), lambda b, i: (b, i, 0, 0)),
        pl.BlockSpec((1, BI, N, R), lambda b, i: (b, i, 0, 0)),
        pl.BlockSpec((1, BI, N, 1), lambda b, i: (b, i, 0, 0)),
        pl.BlockSpec((1, BI, N, 1), lambda b, i: (b, i, 0, 0)),
        pl.BlockSpec((1, BI, N, 1), lambda b, i: (b, i, 0, 0)),
        pl.BlockSpec((1, BI, KP), lambda b, i: (b, i, 0)),
        pl.BlockSpec((1, BI, KP), lambda b, i: (b, i, 0)),
        pl.BlockSpec((1, BI, KP), lambda b, i: (b, i, 0)),
        pl.BlockSpec((1, BI, KP), lambda b, i: (b, i, 0)),
        pl.BlockSpec((1, BI, KP), lambda b, i: (b, i, 0)),
        pl.BlockSpec((1, BI, KP * R), lambda b, i: (b, i, 0)),
        pl.BlockSpec((1, BI, KP * D), lambda b, i: (b, i, 0)),
    ]
    in_specs = row_specs + [full(a) for a in args[13:]]

    nd, es = pl.pallas_call(
        _fused_kernel,
        grid=(B, N // BI),
        in_specs=in_specs,
        out_specs=[
            pl.BlockSpec((1, BI, D), lambda b, i: (b, i, 0)),
            pl.BlockSpec((1, BI, D), lambda b, i: (b, i, 0)),
        ],
        out_shape=[
            jax.ShapeDtypeStruct((B, N, D), f32),
            jax.ShapeDtypeStruct((B, N, D), f32),
        ],
    )(*args)

    denom = jnp.maximum(maskf.sum(axis=(1, 2)), 1.0)[:, None]
    bond_graph = es.sum(axis=1) / denom
    return nd, bond_graph


# speedup vs baseline: 1.0588x; 1.0588x over previous
import functools

import jax
import jax.numpy as jnp
from jax import lax
from jax.experimental import pallas as pl
from jax.experimental.pallas import tpu as pltpu
from jax.experimental.pallas import tpu_sc as plsc

B, N, D, R, K, ORDER, H = 2, 96, 64, 32, 8, 3, 64
BI = 16
NW = 32
KP = 16
NEG = -1e30


def _silu(x):
    return x * jax.nn.sigmoid(x)


def _sc_topk(md, mk, rxf, ryf, rzf, rbf, node_s):
    apw = (B * N) // NW
    nc = N // 16
    mesh = plsc.VectorSubcoreMesh(core_axis_name="c", subcore_axis_name="s")
    f32 = jnp.float32
    i32 = jnp.int32

    @functools.partial(
        pl.kernel, mesh=mesh,
        out_type=[jax.ShapeDtypeStruct((B * N * KP,), i32)]
        + [jax.ShapeDtypeStruct((B * N * KP,), f32)] * 4
        + [jax.ShapeDtypeStruct((B * N * KP * R,), f32),
           jax.ShapeDtypeStruct((B * N * KP * D,), f32)],
        scratch_types=[pltpu.VMEM((N,), f32)] * 5
        + [pltpu.VMEM((KP,), i32)] + [pltpu.VMEM((KP,), f32)] * 4
        + [pltpu.VMEM((N * R,), f32), pltpu.VMEM((N * D,), f32),
           pltpu.VMEM((KP * R,), f32), pltpu.VMEM((KP * D,), f32)],
        compiler_params=pltpu.CompilerParams(needs_layout_passes=False),
    )
    def k(md_h, mk_h, rx_h, ry_h, rz_h, rbf_h, nd_h,
          oi_h, om_h, ox_h, oy_h, oz_h, orb_h, ond_h,
          md_v, mk_v, rx_v, ry_v, rz_v, oi_v, om_v, ox_v, oy_v, oz_v,
          rba_v, nda_v, rb_v, nd_v):
        wid = lax.axis_index("s") * 2 + lax.axis_index("c")
        iota = lax.iota(i32, 16)
        bw = (wid * apw) // N
        pltpu.sync_copy(nd_h.at[pl.ds(bw * N * D, N * D)], nda_v)

        def body(t, carry):
            a = wid * apw + t
            roff = a * N
            pltpu.sync_copy(md_h.at[pl.ds(roff, N)], md_v)
            pltpu.sync_copy(mk_h.at[pl.ds(roff, N)], mk_v)
            pltpu.sync_copy(rx_h.at[pl.ds(roff, N)], rx_v)
            pltpu.sync_copy(ry_h.at[pl.ds(roff, N)], ry_v)
            pltpu.sync_copy(rz_h.at[pl.ds(roff, N)], rz_v)
            v = [md_v[pl.ds(c * 16, 16)] for c in range(nc)]
            oidx = jnp.zeros((16,), i32)
            for r in range(K):
                m = v[0]
                for c in range(1, nc):
                    m = jnp.minimum(m, v[c])
                s = jnp.min(m)
                best = jnp.full((16,), 32767, i32)
                for c in range(nc):
                    eq = v[c] == s
                    pop = plsc.all_reduce_population_count(eq)
                    ffs = plsc.all_reduce_ffs(eq)
                    cand = jnp.where(pop > 0, c * 16 + ffs, 32767)
                    best = jnp.minimum(best, cand)
                oidx = jnp.where(iota == r, best, oidx)
                for c in range(nc):
                    v[c] = jnp.where(c * 16 + iota == best, 1e9, v[c])
            oi_v[...] = oidx
            om_v[...] = plsc.load_gather(mk_v, [oidx])
            ox_v[...] = plsc.load_gather(rx_v, [oidx])
            oy_v[...] = plsc.load_gather(ry_v, [oidx])
            oz_v[...] = plsc.load_gather(rz_v, [oidx])
            pltpu.sync_copy(rbf_h.at[pl.ds(a * N * R, N * R)], rba_v)
            o32 = oidx * R
            o64 = oidx * D
            for g in range(K * R // 16):
                base = o32.at[jnp.full((16,), g // (R // 16), i32)].get(
                    mode='promise_in_bounds')
                fi = base + (g % (R // 16)) * 16 + iota
                rb_v[pl.ds(g * 16, 16)] = plsc.load_gather(rba_v, [fi])
            for g in range(K * D // 16):
                base = o64.at[jnp.full((16,), g // (D // 16), i32)].get(
                    mode='promise_in_bounds')
                fi = base + (g % (D // 16)) * 16 + iota
                nd_v[pl.ds(g * 16, 16)] = plsc.load_gather(nda_v, [fi])
            ooff = a * KP
            pltpu.sync_copy(oi_v, oi_h.at[pl.ds(ooff, KP)])
            pltpu.sync_copy(om_v, om_h.at[pl.ds(ooff, KP)])
            pltpu.sync_copy(ox_v, ox_h.at[pl.ds(ooff, KP)])
            pltpu.sync_copy(oy_v, oy_h.at[pl.ds(ooff, KP)])
            pltpu.sync_copy(oz_v, oz_h.at[pl.ds(ooff, KP)])
            pltpu.sync_copy(rb_v, orb_h.at[pl.ds(a * KP * R, KP * R)])
            pltpu.sync_copy(nd_v, ond_h.at[pl.ds(a * KP * D, KP * D)])
            return carry

        lax.fori_loop(0, apw, body, 0)

    oi, om, ox, oy, oz, orb, ond = k(
        md.reshape(-1), mk.reshape(-1),
        rxf.reshape(-1), ryf.reshape(-1), rzf.reshape(-1),
        rbf.reshape(-1), node_s.reshape(-1))
    rs = (B, N, KP)
    return (oi.reshape(rs), om.reshape(rs), ox.reshape(rs),
            oy.reshape(rs), oz.reshape(rs),
            orb.reshape(B, N, KP * R), ond.reshape(B, N, KP * D))


def _fused_kernel(node_ref, mask_ref, rbf_ref, rx_ref, ry_ref, rz_ref,
                  oi_ref, om_ref, ox_ref, oy_ref, oz_ref, orb_ref, ond_ref,
                  ep_w1, ep_b1, ep_w2, ep_b2,
                  ktp, c6, cb0, tw2_2, tpb2t,
                  ts1_2, tsb1t, kts, w2p, ts_b2, bc,
                  tm_g, tm_b, tm_w1, tm_b1, tm_w2, tm_b2,
                  eg_w, eg_b, no_g, no_b, no_w, no_b2, en_g, en_b,
                  nd_out, es_out):
    f32 = jnp.float32
    i_blk = pl.program_id(1)
    node_b = node_ref[0]
    maskf = mask_ref[0]
    rbf = rbf_ref[0]
    rx = rx_ref[0]
    ry = ry_ref[0]
    rz = rz_ref[0]

    idx = oi_ref[0][:, :K]
    tmask = om_ref[0][:, :K]
    crx = ox_ref[0][:, :K]
    cry = oy_ref[0][:, :K]
    crz = oz_ref[0][:, :K]
    crbf = orb_ref[0]
    cnd = ond_ref[0]

    maskC = maskf[:, :, None]
    cos_t = (crx[:, :, None] * rx[:, None, :] +
             cry[:, :, None] * ry[:, None, :] +
             crz[:, :, None] * rz[:, None, :])
    cos_t = jnp.clip(cos_t, -1.0 + 1e-6, 1.0 - 1e-6)
    cosT = jnp.transpose(cos_t, (0, 2, 1))

    logit_parts = []
    twcf_parts = []
    for p in range(K // 2):
        s0 = 2 * p
        cosp = cosT[:, :, s0:s0 + 2]
        c2p = cosp * cosp
        c3p = c2p * cosp
        xp = jnp.concatenate([cosp, c2p, c3p], axis=2)
        radp = crbf[:, 2 * R * p:2 * R * (p + 1)] @ ktp[...]
        bslice = radp + cb0[:, 128 * p:128 * (p + 1)]
        thp = (xp.reshape(BI * N, 6) @ c6[...]).reshape(BI, N, 2 * H) \
            + bslice[:, None, :]
        thp = _silu(thp)
        twp = thp.reshape(BI * N, 2 * H) @ tw2_2[...] + tpb2t[...]
        cfp = cnd[:, 128 * p:128 * (p + 1)]
        cfs = cfp @ kts[...]
        shp = _silu((twp @ ts1_2[...] + tsb1t[...]).reshape(BI, N, 2 * H)
                    + cfs[:, None, :])
        logit_parts.append(shp.reshape(BI * N, 2 * H) @ w2p[...])
        twcf_parts.append(twp.reshape(BI, N, 2 * H) * cfp[:, None, :])

    logits = jnp.concatenate(logit_parts, axis=1).reshape(BI, N, K) \
        + ts_b2[...].reshape(1, 1, 1)

    iota_jn = jax.lax.broadcasted_iota(jnp.int32, (BI, N, K), 1)
    pm = maskC * tmask[:, None, :]
    pm = jnp.where(idx[:, None, :] == iota_jn, 0.0, pm)

    logits = jnp.where(pm <= 0.0, NEG, logits)
    lmax = jnp.max(logits, axis=2, keepdims=True)
    e = jnp.exp(logits - lmax)
    attn = e / jnp.sum(e, axis=2, keepdims=True)
    attn = jnp.where(pm > 0.0, attn, 0.0)

    attnb = attn.reshape(BI * N, K) @ bc[...]
    pmb = pm.reshape(BI * N, K) @ bc[...]
    attnb = attnb.reshape(BI, N, K * D)
    pmb = pmb.reshape(BI, N, K * D)

    t_attn = jnp.zeros((BI, N, D), f32)
    t_max = jnp.full((BI, N, D), NEG, f32)
    for p in range(K // 2):
        twcf = twcf_parts[p]
        wp = twcf * attnb[:, :, 128 * p:128 * (p + 1)]
        t_attn = t_attn + wp[:, :, :D] + wp[:, :, D:]
        mp = jnp.where(pmb[:, :, 128 * p:128 * (p + 1)] <= 0.0, NEG, twcf)
        t_max = jnp.maximum(t_max, jnp.maximum(mp[:, :, :D], mp[:, :, D:]))
    t_max = jnp.where(t_max <= NEG * 0.5, 0.0, t_max)

    mi = jnp.concatenate([t_attn, t_max], axis=2)
    mu = jnp.mean(mi, axis=2, keepdims=True)
    mv = jnp.mean((mi - mu) ** 2, axis=2, keepdims=True)
    mi = (mi - mu) * (1.0 / jnp.sqrt(mv + 1e-5)) * tm_g[...].reshape(1, 1, 2 * D) \
        + tm_b[...].reshape(1, 1, 2 * D)
    mh = _silu(mi.reshape(BI * N, 2 * D) @ tm_w1[...] + tm_b1[...])
    ctx = mh @ tm_w2[...] + tm_b2[...]

    node_i = node_ref[0, pl.ds(i_blk * BI, BI), :]
    hi = node_i @ ep_w1[:D, :]
    dstW = node_b @ ep_w1[D:2 * D, :]
    rbfW1 = (rbf.reshape(BI * N, R) @ ep_w1[2 * D:, :]).reshape(BI, N, H)
    eh = _silu(hi[:, None, :] + dstW[None, :, :] + rbfW1
               + ep_b1[...].reshape(1, 1, H))
    eb = (eh.reshape(BI * N, H) @ ep_w2[...] + ep_b2[...]).reshape(BI, N, D)
    mask3 = maskC
    eb = eb * mask3

    ef = eb + ctx.reshape(BI, N, D)
    emu = jnp.mean(ef, axis=2, keepdims=True)
    ev = jnp.mean((ef - emu) ** 2, axis=2, keepdims=True)
    ef = (ef - emu) * (1.0 / jnp.sqrt(ev + 1e-5)) * en_g[...].reshape(1, 1, D) \
        + en_b[...].reshape(1, 1, D)
    gate = jax.nn.sigmoid(ef.reshape(BI * N, D) @ eg_w[...]
                          + eg_b[...]).reshape(BI, N, D)
    ef = gate * ef

    ns = jnp.sum(ef * mask3, axis=1)
    es = jnp.sum(ef, axis=1)

    nmu = jnp.mean(ns, axis=1, keepdims=True)
    nv = jnp.mean((ns - nmu) ** 2, axis=1, keepdims=True)
    nd = (ns - nmu) * (1.0 / jnp.sqrt(nv + 1e-5)) * no_g[...] + no_b[...]
    nd = nd @ no_w[...] + no_b2[...]

    nd_out[0] = nd
    es_out[0] = es


def kernel(node_s, dist, rbf, r_hat, access_mask, params):
    p = params
    f32 = jnp.float32
    maskf = access_mask.astype(f32)
    maxd = jnp.maximum(dist.max(axis=(1, 2), keepdims=True), 1.0) + 1.0
    md = jnp.where(access_mask, dist, maxd)
    rx = r_hat[..., 0]
    ry = r_hat[..., 1]
    rz = r_hat[..., 2]

    oi, om, ox, oy, oz, orb, ond = _sc_topk(md, maskf, rx, ry, rz, rbf, node_s)

    def row2(v):
        return v.reshape(1, -1)

    w = p['tp_w1']
    tp_c = jnp.stack([w[0] - 0.5 * w[2], w[1] - 1.5 * w[3],
                      1.5 * w[2], 2.5 * w[3]], axis=0)
    eye2 = jnp.eye(2, dtype=f32)
    eyeK = jnp.eye(K, dtype=f32)
    c6 = jnp.einsum('qr,th->tqrh', eye2, tp_c[1:]).reshape(6, 2 * H)
    cb0 = jnp.tile(tp_c[0] + p['tp_b1'], 2 * K // 2).reshape(1, K * H)
    tw2_2 = jnp.einsum('qr,hd->qhrd', eye2, p['tp_w2']).reshape(2 * H, 2 * H)
    tpb2t = jnp.tile(p['tp_b2'], 2).reshape(1, 2 * H)
    ts1_2 = jnp.einsum('qr,dh->qdrh', eye2, p['ts_w1'][:D]).reshape(2 * D, 2 * H)
    tsb1t = jnp.tile(p['ts_b1'], 2).reshape(1, 2 * H)
    w2p = jnp.einsum('qr,ho->qhro', eye2, p['ts_w2']).reshape(2 * H, 2)
    bc = jnp.einsum('kl,d->kld', eyeK, jnp.ones((D,), f32)).reshape(K, K * D)
    ktp = jnp.einsum('qr,ah->qarh', eye2,
                     p['tp_w1'][ORDER + 1:]).reshape(2 * R, 2 * H)
    kts = jnp.einsum('qr,dh->qdrh', eye2, p['ts_w1'][D:]).reshape(2 * D, 2 * H)

    args = (node_s, maskf, rbf, rx, ry, rz, oi, om, ox, oy, oz, orb, ond,
            p['ep_w1'], row2(p['ep_b1']), p['ep_w2'], row2(p['ep_b2']),
            ktp, c6, cb0, tw2_2, tpb2t,
            ts1_2, tsb1t, kts, w2p, row2(p['ts_b2']), bc,
            row2(p['tm_g']), row2(p['tm_b']),
            p['tm_w1'], row2(p['tm_b1']), p['tm_w2'], row2(p['tm_b2']),
            p['eg_w'], row2(p['eg_b']), row2(p['no_g']), row2(p['no_b']),
            p['no_w'], row2(p['no_b2']), row2(p['en_g']), row2(p['en_b']))

    def full(a):
        return pl.BlockSpec(a.shape, lambda b, i: (0,) * a.ndim)

    row_specs = [
        pl.BlockSpec((1, N, D), lambda b, i: (b, 0, 0)),
        pl.BlockSpec((1, BI, N), lambda b, i: (b, i, 0)),
        pl.BlockSpec((1, BI, N, R), lambda b, i: (b, i, 0, 0)),
        pl.BlockSpec((1, BI, N), lambda b, i: (b, i, 0)),
        pl.BlockSpec((1, BI, N), lambda b, i: (b, i, 0)),
        pl.BlockSpec((1, BI, N), lambda b, i: (b, i, 0)),
        pl.BlockSpec((1, BI, KP), lambda b, i: (b, i, 0)),
        pl.BlockSpec((1, BI, KP), lambda b, i: (b, i, 0)),
        pl.BlockSpec((1, BI, KP), lambda b, i: (b, i, 0)),
        pl.BlockSpec((1, BI, KP), lambda b, i: (b, i, 0)),
        pl.BlockSpec((1, BI, KP), lambda b, i: (b, i, 0)),
        pl.BlockSpec((1, BI, KP * R), lambda b, i: (b, i, 0)),
        pl.BlockSpec((1, BI, KP * D), lambda b, i: (b, i, 0)),
    ]
    in_specs = row_specs + [full(a) for a in args[13:]]

    nd, es = pl.pallas_call(
        _fused_kernel,
        grid=(B, N // BI),
        in_specs=in_specs,
        out_specs=[
            pl.BlockSpec((1, BI, D), lambda b, i: (b, i, 0)),
            pl.BlockSpec((1, BI, D), lambda b, i: (b, i, 0)),
        ],
        out_shape=[
            jax.ShapeDtypeStruct((B, N, D), f32),
            jax.ShapeDtypeStruct((B, N, D), f32),
        ],
    )(*args)

    denom = jnp.maximum(maskf.sum(axis=(1, 2)), 1.0)[:, None]
    bond_graph = es.sum(axis=1) / denom
    return nd, bond_graph


# speedup vs baseline: 1.1807x; 1.1152x over previous
import functools

import jax
import jax.numpy as jnp
from jax import lax
from jax.experimental import pallas as pl
from jax.experimental.pallas import tpu as pltpu
from jax.experimental.pallas import tpu_sc as plsc

B, N, D, R, K, ORDER, H = 2, 96, 64, 32, 8, 3, 64
BI = 16
NW = 32
KP = 16
NEG = -1e30


def _silu(x):
    return x * jax.nn.sigmoid(x)


OB = 896


def _sc_topk(sstack, rbf, node_s):
    apw = (B * N) // NW
    nc = N // 16
    mesh = plsc.VectorSubcoreMesh(core_axis_name="c", subcore_axis_name="s")
    f32 = jnp.float32
    i32 = jnp.int32

    @functools.partial(
        pl.kernel, mesh=mesh,
        out_type=jax.ShapeDtypeStruct((B * N * OB,), f32),
        scratch_types=[pltpu.VMEM((5 * N,), f32),
                       pltpu.VMEM((N * R,), f32), pltpu.VMEM((N * D,), f32),
                       pltpu.VMEM((OB,), f32)],
        compiler_params=pltpu.CompilerParams(needs_layout_passes=False),
    )
    def k(ss_h, rbf_h, nd_h, ob_h, ss_v, rba_v, nda_v, ob_v):
        wid = lax.axis_index("s") * 2 + lax.axis_index("c")
        iota = lax.iota(i32, 16)
        bw = (wid * apw) // N
        pltpu.sync_copy(nd_h.at[pl.ds(bw * N * D, N * D)], nda_v)

        def body(t, carry):
            a = wid * apw + t
            pltpu.sync_copy(ss_h.at[pl.ds(a * 5 * N, 5 * N)], ss_v)
            pltpu.sync_copy(rbf_h.at[pl.ds(a * N * R, N * R)], rba_v)
            v = [ss_v[pl.ds(c * 16, 16)] for c in range(nc)]
            oidx = jnp.zeros((16,), i32)
            for r in range(K):
                m = v[0]
                for c in range(1, nc):
                    m = jnp.minimum(m, v[c])
                s = jnp.min(m)
                best = jnp.full((16,), 32767, i32)
                for c in range(nc):
                    eq = v[c] == s
                    pop = plsc.all_reduce_population_count(eq)
                    ffs = plsc.all_reduce_ffs(eq)
                    cand = jnp.where(pop > 0, c * 16 + ffs, 32767)
                    best = jnp.minimum(best, cand)
                oidx = jnp.where(iota == r, best, oidx)
                for c in range(nc):
                    v[c] = jnp.where(c * 16 + iota == best, 1e9, v[c])
            ob_v[pl.ds(0, 16)] = oidx.astype(f32)
            for q in range(4):
                ob_v[pl.ds(16 * (q + 1), 16)] = plsc.load_gather(
                    ss_v, [oidx + (q + 1) * N])
            o32 = oidx * R
            o64 = oidx * D
            for g in range(K * R // 16):
                base = o32.at[jnp.full((16,), g // (R // 16), i32)].get(
                    mode='promise_in_bounds')
                fi = base + (g % (R // 16)) * 16 + iota
                ob_v[pl.ds(128 + g * 16, 16)] = plsc.load_gather(rba_v, [fi])
            for g in range(K * D // 16):
                base = o64.at[jnp.full((16,), g // (D // 16), i32)].get(
                    mode='promise_in_bounds')
                fi = base + (g % (D // 16)) * 16 + iota
                ob_v[pl.ds(384 + g * 16, 16)] = plsc.load_gather(nda_v, [fi])
            pltpu.sync_copy(ob_v, ob_h.at[pl.ds(a * OB, OB)])
            return carry

        lax.fori_loop(0, apw, body, 0)

    ob = k(sstack.reshape(-1), rbf.reshape(-1), node_s.reshape(-1))
    return ob.reshape(B, N, OB)


def _fused_kernel(node_ref, mask_ref, rbf_ref, rx_ref, ry_ref, rz_ref,
                  ob_ref,
                  ep_w1, ep_b1, ep_w2, ep_b2,
                  ktp, c6, cb0, tw2_2, tpb2t,
                  ts1_2, tsb1t, kts, w2p, ts_b2, bc,
                  tm_g, tm_b, tm_w1, tm_b1, tm_w2, tm_b2,
                  eg_w, eg_b, no_g, no_b, no_w, no_b2, en_g, en_b,
                  nd_out, es_out):
    f32 = jnp.float32
    i_blk = pl.program_id(1)
    node_b = node_ref[0]
    maskf = mask_ref[0]
    rbf = rbf_ref[0]
    rx = rx_ref[0]
    ry = ry_ref[0]
    rz = rz_ref[0]

    obuf = ob_ref[0]
    idx = obuf[:, 0:K].astype(jnp.int32)
    tmask = obuf[:, 16:16 + K]
    crx = obuf[:, 32:32 + K]
    cry = obuf[:, 48:48 + K]
    crz = obuf[:, 64:64 + K]

    maskC = maskf[:, :, None]
    cos_t = (crx[:, :, None] * rx[:, None, :] +
             cry[:, :, None] * ry[:, None, :] +
             crz[:, :, None] * rz[:, None, :])
    cos_t = jnp.clip(cos_t, -1.0 + 1e-6, 1.0 - 1e-6)
    cosT = jnp.transpose(cos_t, (0, 2, 1))

    logit_parts = []
    twcf_parts = []
    for p in range(K // 2):
        s0 = 2 * p
        cosp = cosT[:, :, s0:s0 + 2]
        c2p = cosp * cosp
        c3p = c2p * cosp
        xp = jnp.concatenate([cosp, c2p, c3p], axis=2)
        radp = obuf[:, 128 + 2 * R * p:128 + 2 * R * (p + 1)] @ ktp[...]
        bslice = radp + cb0[:, 128 * p:128 * (p + 1)]
        thp = (xp.reshape(BI * N, 6) @ c6[...]).reshape(BI, N, 2 * H) \
            + bslice[:, None, :]
        thp = _silu(thp)
        twp = thp.reshape(BI * N, 2 * H) @ tw2_2[...] + tpb2t[...]
        cfp = obuf[:, 384 + 128 * p:384 + 128 * (p + 1)]
        cfs = cfp @ kts[...]
        shp = _silu((twp @ ts1_2[...] + tsb1t[...]).reshape(BI, N, 2 * H)
                    + cfs[:, None, :])
        logit_parts.append(shp.reshape(BI * N, 2 * H) @ w2p[...])
        twcf_parts.append(twp.reshape(BI, N, 2 * H) * cfp[:, None, :])

    logits = jnp.concatenate(logit_parts, axis=1).reshape(BI, N, K) \
        + ts_b2[...].reshape(1, 1, 1)

    iota_jn = jax.lax.broadcasted_iota(jnp.int32, (BI, N, K), 1)
    pm = maskC * tmask[:, None, :]
    pm = jnp.where(idx[:, None, :] == iota_jn, 0.0, pm)

    logits = jnp.where(pm <= 0.0, NEG, logits)
    lmax = jnp.max(logits, axis=2, keepdims=True)
    e = jnp.exp(logits - lmax)
    attn = e / jnp.sum(e, axis=2, keepdims=True)
    attn = jnp.where(pm > 0.0, attn, 0.0)

    attnb = attn.reshape(BI * N, K) @ bc[...]
    pmb = pm.reshape(BI * N, K) @ bc[...]
    attnb = attnb.reshape(BI, N, K * D)
    pmb = pmb.reshape(BI, N, K * D)

    t_attn = jnp.zeros((BI, N, D), f32)
    t_max = jnp.full((BI, N, D), NEG, f32)
    for p in range(K // 2):
        twcf = twcf_parts[p]
        wp = twcf * attnb[:, :, 128 * p:128 * (p + 1)]
        t_attn = t_attn + wp[:, :, :D] + wp[:, :, D:]
        mp = jnp.where(pmb[:, :, 128 * p:128 * (p + 1)] <= 0.0, NEG, twcf)
        t_max = jnp.maximum(t_max, jnp.maximum(mp[:, :, :D], mp[:, :, D:]))
    t_max = jnp.where(t_max <= NEG * 0.5, 0.0, t_max)

    mi = jnp.concatenate([t_attn, t_max], axis=2)
    mu = jnp.mean(mi, axis=2, keepdims=True)
    mv = jnp.mean((mi - mu) ** 2, axis=2, keepdims=True)
    mi = (mi - mu) * (1.0 / jnp.sqrt(mv + 1e-5)) * tm_g[...].reshape(1, 1, 2 * D) \
        + tm_b[...].reshape(1, 1, 2 * D)
    mh = _silu(mi.reshape(BI * N, 2 * D) @ tm_w1[...] + tm_b1[...])
    ctx = mh @ tm_w2[...] + tm_b2[...]

    node_i = node_ref[0, pl.ds(i_blk * BI, BI), :]
    hi = node_i @ ep_w1[:D, :]
    dstW = node_b @ ep_w1[D:2 * D, :]
    rbfW1 = (rbf.reshape(BI * N, R) @ ep_w1[2 * D:, :]).reshape(BI, N, H)
    eh = _silu(hi[:, None, :] + dstW[None, :, :] + rbfW1
               + ep_b1[...].reshape(1, 1, H))
    eb = (eh.reshape(BI * N, H) @ ep_w2[...] + ep_b2[...]).reshape(BI, N, D)
    mask3 = maskC
    eb = eb * mask3

    ef = eb + ctx.reshape(BI, N, D)
    emu = jnp.mean(ef, axis=2, keepdims=True)
    ev = jnp.mean((ef - emu) ** 2, axis=2, keepdims=True)
    ef = (ef - emu) * (1.0 / jnp.sqrt(ev + 1e-5)) * en_g[...].reshape(1, 1, D) \
        + en_b[...].reshape(1, 1, D)
    gate = jax.nn.sigmoid(ef.reshape(BI * N, D) @ eg_w[...]
                          + eg_b[...]).reshape(BI, N, D)
    ef = gate * ef

    ns = jnp.sum(ef * mask3, axis=1)
    es = jnp.sum(ef, axis=1)

    nmu = jnp.mean(ns, axis=1, keepdims=True)
    nv = jnp.mean((ns - nmu) ** 2, axis=1, keepdims=True)
    nd = (ns - nmu) * (1.0 / jnp.sqrt(nv + 1e-5)) * no_g[...] + no_b[...]
    nd = nd @ no_w[...] + no_b2[...]

    nd_out[0] = nd
    es_out[0] = es


def kernel(node_s, dist, rbf, r_hat, access_mask, params):
    p = params
    f32 = jnp.float32
    maskf = access_mask.astype(f32)
    maxd = jnp.maximum(dist.max(axis=(1, 2), keepdims=True), 1.0) + 1.0
    md = jnp.where(access_mask, dist, maxd)
    rx = r_hat[..., 0]
    ry = r_hat[..., 1]
    rz = r_hat[..., 2]

    sstack = jnp.stack([md, maskf, rx, ry, rz], axis=2)
    obuf = _sc_topk(sstack, rbf, node_s)

    def row2(v):
        return v.reshape(1, -1)

    w = p['tp_w1']
    tp_c = jnp.stack([w[0] - 0.5 * w[2], w[1] - 1.5 * w[3],
                      1.5 * w[2], 2.5 * w[3]], axis=0)
    eye2 = jnp.eye(2, dtype=f32)
    eyeK = jnp.eye(K, dtype=f32)
    c6 = jnp.einsum('qr,th->tqrh', eye2, tp_c[1:]).reshape(6, 2 * H)
    cb0 = jnp.tile(tp_c[0] + p['tp_b1'], 2 * K // 2).reshape(1, K * H)
    tw2_2 = jnp.einsum('qr,hd->qhrd', eye2, p['tp_w2']).reshape(2 * H, 2 * H)
    tpb2t = jnp.tile(p['tp_b2'], 2).reshape(1, 2 * H)
    ts1_2 = jnp.einsum('qr,dh->qdrh', eye2, p['ts_w1'][:D]).reshape(2 * D, 2 * H)
    tsb1t = jnp.tile(p['ts_b1'], 2).reshape(1, 2 * H)
    w2p = jnp.einsum('qr,ho->qhro', eye2, p['ts_w2']).reshape(2 * H, 2)
    bc = jnp.einsum('kl,d->kld', eyeK, jnp.ones((D,), f32)).reshape(K, K * D)
    ktp = jnp.einsum('qr,ah->qarh', eye2,
                     p['tp_w1'][ORDER + 1:]).reshape(2 * R, 2 * H)
    kts = jnp.einsum('qr,dh->qdrh', eye2, p['ts_w1'][D:]).reshape(2 * D, 2 * H)

    args = (node_s, maskf, rbf, rx, ry, rz, obuf,
            p['ep_w1'], row2(p['ep_b1']), p['ep_w2'], row2(p['ep_b2']),
            ktp, c6, cb0, tw2_2, tpb2t,
            ts1_2, tsb1t, kts, w2p, row2(p['ts_b2']), bc,
            row2(p['tm_g']), row2(p['tm_b']),
            p['tm_w1'], row2(p['tm_b1']), p['tm_w2'], row2(p['tm_b2']),
            p['eg_w'], row2(p['eg_b']), row2(p['no_g']), row2(p['no_b']),
            p['no_w'], row2(p['no_b2']), row2(p['en_g']), row2(p['en_b']))

    def full(a):
        return pl.BlockSpec(a.shape, lambda b, i: (0,) * a.ndim)

    row_specs = [
        pl.BlockSpec((1, N, D), lambda b, i: (b, 0, 0)),
        pl.BlockSpec((1, BI, N), lambda b, i: (b, i, 0)),
        pl.BlockSpec((1, BI, N, R), lambda b, i: (b, i, 0, 0)),
        pl.BlockSpec((1, BI, N), lambda b, i: (b, i, 0)),
        pl.BlockSpec((1, BI, N), lambda b, i: (b, i, 0)),
        pl.BlockSpec((1, BI, N), lambda b, i: (b, i, 0)),
        pl.BlockSpec((1, BI, OB), lambda b, i: (b, i, 0)),
    ]
    in_specs = row_specs + [full(a) for a in args[7:]]

    nd, es = pl.pallas_call(
        _fused_kernel,
        grid=(B, N // BI),
        in_specs=in_specs,
        out_specs=[
            pl.BlockSpec((1, BI, D), lambda b, i: (b, i, 0)),
            pl.BlockSpec((1, BI, D), lambda b, i: (b, i, 0)),
        ],
        out_shape=[
            jax.ShapeDtypeStruct((B, N, D), f32),
            jax.ShapeDtypeStruct((B, N, D), f32),
        ],
    )(*args)

    denom = jnp.maximum(maskf.sum(axis=(1, 2)), 1.0)[:, None]
    bond_graph = es.sum(axis=1) / denom
    return nd, bond_graph
